# Initial kernel scaffold; baseline (speedup 1.0000x reference)
import jax
import jax.numpy as jnp
from jax.experimental import pallas as pl


def kernel(atom_fea, nbr_fea, nbr_fea_idx, crystal_atom_idx, W_emb, b_emb, Wf, bf, g1, be1, g2, be2, W_fc, b_fc, W_out, b_out):
    raise NotImplementedError("write your pallas kernel here")


# speedup vs baseline: 2.0281x; 2.0281x over previous
import functools

import jax
import jax.numpy as jnp
from jax import lax
from jax.experimental import pallas as pl
from jax.experimental.pallas import tpu as pltpu
from jax.experimental.pallas import tpu_sc as plsc

EPS = 1e-5


def _softplus(x):
    m = jnp.maximum(x, 0.0)
    return m + jnp.log(jnp.exp(x - m) + jnp.exp(-m))


def _sigmoid(x):
    return 1.0 / (1.0 + jnp.exp(-x))


_CH = 128


@functools.lru_cache(maxsize=None)
def _make_sc_gather(V, D, B):
    info = plsc.get_sparse_core_info()
    NW = info.num_cores * info.num_subcores
    assert B % (8 * NW) == 0
    b_per_w = B // NW
    ch = min(_CH, b_per_w)
    n_full = b_per_w // ch
    rem = b_per_w - n_full * ch

    mesh = plsc.VectorSubcoreMesh(core_axis_name="c", subcore_axis_name="s")
    scratch = [
        pltpu.VMEM((ch,), jnp.int32),
        pltpu.VMEM((ch, D), jnp.float32),
        pltpu.SemaphoreType.DMA,
    ]
    if rem:
        scratch += [
            pltpu.VMEM((rem,), jnp.int32),
            pltpu.VMEM((rem, D), jnp.float32),
        ]

    @functools.partial(
        pl.kernel, mesh=mesh,
        out_type=jax.ShapeDtypeStruct((B, D), jnp.float32),
        scratch_types=scratch,
        compiler_params=pltpu.CompilerParams(use_tc_tiling_on_sc=False),
    )
    def gather_kernel(table_hbm, idx_hbm, out_hbm, idx_v, rows_v, sem,
                      *rem_scratch):
        wid = lax.axis_index("s") * info.num_cores + lax.axis_index("c")
        base = wid * b_per_w

        def do_chunk(off, idx_ref, rows_ref, size):
            pltpu.sync_copy(idx_hbm.at[pl.ds(off, size)], idx_ref)
            pltpu.async_copy(table_hbm.at[idx_ref], rows_ref, sem).wait()
            pltpu.sync_copy(rows_ref, out_hbm.at[pl.ds(off, size)])

        def body(i, carry):
            do_chunk(base + i * ch, idx_v, rows_v, ch)
            return carry

        lax.fori_loop(0, n_full, body, 0)
        if rem:
            idx_r, rows_r = rem_scratch
            do_chunk(base + n_full * ch, idx_r, rows_r, rem)

    return gather_kernel


def _sc_gather(table, idx):
    V, D = table.shape
    (B,) = idx.shape
    return _make_sc_gather(V, D, B)(table, idx)


def _embed_body(x_ref, w_ref, b_ref, o_ref):
    o_ref[...] = (
        jnp.dot(x_ref[...], w_ref[...], preferred_element_type=jnp.float32)
        + b_ref[...]
    )


def _embed(atom_fea, W_emb, b_emb):
    N, orig = atom_fea.shape
    d = W_emb.shape[1]
    BN = 2000
    return pl.pallas_call(
        _embed_body,
        grid=(N // BN,),
        in_specs=[
            pl.BlockSpec((BN, orig), lambda i: (i, 0)),
            pl.BlockSpec((orig, d), lambda i: (0, 0)),
            pl.BlockSpec((1, d), lambda i: (0, 0)),
        ],
        out_specs=pl.BlockSpec((BN, d), lambda i: (i, 0)),
        out_shape=jax.ShapeDtypeStruct((N, d), jnp.float32),
    )(atom_fea, W_emb, b_emb.reshape(1, d))


def _conv_body(BN, M, h_ref, hg_ref, nf_ref, wsf_ref, wsc_ref, wnf_ref,
               wnc_ref, wef_ref, wec_ref, bf_ref, bc_ref, g2_ref, be2_ref,
               o_ref):
    h = h_ref[...]
    hg = hg_ref[...]
    nf = nf_ref[...]
    d = h.shape[1]
    dot = functools.partial(jnp.dot, preferred_element_type=jnp.float32)
    sF = dot(h, wsf_ref[...]) + bf_ref[...]
    sC = dot(h, wsc_ref[...]) + bc_ref[...]
    eF = dot(hg, wnf_ref[...]) + dot(nf, wef_ref[...])
    eC = dot(hg, wnc_ref[...]) + dot(nf, wec_ref[...])
    F = eF.reshape(BN, M, d) + sF[:, None, :]
    C = eC.reshape(BN, M, d) + sC[:, None, :]
    msg = jnp.sum(_sigmoid(F) * _softplus(C), axis=1)
    o_ref[...] = _softplus(h + msg * g2_ref[...] + be2_ref[...])


def _conv_layer(h, hg, nf2, Wl, bl, g2l, be2l):
    N, d = h.shape
    M = hg.shape[0] // N
    e = nf2.shape[1]
    BN = 400
    full = lambda shape: pl.BlockSpec(shape, lambda i: (0, 0))
    args = (
        h, hg, nf2,
        Wl[:d, :d], Wl[:d, d:],
        Wl[d:2 * d, :d], Wl[d:2 * d, d:],
        Wl[2 * d:, :d], Wl[2 * d:, d:],
        bl[:d].reshape(1, d), bl[d:].reshape(1, d),
        g2l.reshape(1, d), be2l.reshape(1, d),
    )
    return pl.pallas_call(
        functools.partial(_conv_body, BN, M),
        grid=(N // BN,),
        in_specs=[
            pl.BlockSpec((BN, d), lambda i: (i, 0)),
            pl.BlockSpec((BN * M, d), lambda i: (i, 0)),
            pl.BlockSpec((BN * M, e), lambda i: (i, 0)),
            full((d, d)), full((d, d)), full((d, d)), full((d, d)),
            full((e, d)), full((e, d)),
            full((1, d)), full((1, d)), full((1, d)), full((1, d)),
        ],
        out_specs=pl.BlockSpec((BN, d), lambda i: (i, 0)),
        out_shape=jax.ShapeDtypeStruct((N, d), jnp.float32),
    )(*args)


def _head_body(A, A_pad, C, pg_ref, wfc_ref, bfc_ref, wout_ref, bout_ref,
               o_ref):
    d = pg_ref.shape[1]
    pg = pg_ref[...].reshape(C, A_pad, d)
    valid = lax.broadcasted_iota(jnp.int32, (C, A_pad, d), 1) < A
    s = jnp.sum(jnp.where(valid, pg, 0.0), axis=1) * (1.0 / A)
    dot = functools.partial(jnp.dot, preferred_element_type=jnp.float32)
    crys = _softplus(dot(s, wfc_ref[...]) + bfc_ref[...])
    o_ref[...] = dot(crys, wout_ref[...]) + bout_ref[...]


def _head(pool_g, C, A, A_pad, W_fc, b_fc, W_out, b_out):
    d = pool_g.shape[1]
    h_fea = W_fc.shape[1]
    full = lambda shape: pl.BlockSpec(shape, lambda: (0, 0))
    return pl.pallas_call(
        functools.partial(_head_body, A, A_pad, C),
        in_specs=[
            full((C * A_pad, d)),
            full((d, h_fea)), full((1, h_fea)),
            full((h_fea, 1)), full((1, 1)),
        ],
        out_specs=full((C, 1)),
        out_shape=jax.ShapeDtypeStruct((C, 1), jnp.float32),
    )(pool_g, W_fc, b_fc.reshape(1, h_fea), W_out, b_out.reshape(1, 1))


def kernel(atom_fea, nbr_fea, nbr_fea_idx, crystal_atom_idx, W_emb, b_emb,
           Wf, bf, g1, be1, g2, be2, W_fc, b_fc, W_out, b_out):
    N, M = nbr_fea_idx.shape
    C, A = crystal_atom_idx.shape
    d = W_emb.shape[1]
    e = nbr_fea.shape[2]
    L = Wf.shape[0]

    inv = 1.0 / jnp.sqrt(1.0 + EPS)
    nf2 = nbr_fea.reshape(N * M, e)
    flat_idx = nbr_fea_idx.reshape(N * M).astype(jnp.int32)

    h = _embed(atom_fea, W_emb, b_emb)
    for l in range(L):
        scale = g1[l] * inv
        Wl = Wf[l] * scale[None, :]
        bl = bf[l] * scale + be1[l]
        hg = _sc_gather(h, flat_idx)
        h = _conv_layer(h, hg, nf2, Wl, bl, g2[l] * inv, be2[l])

    A_pad = 128
    cidx = crystal_atom_idx.astype(jnp.int32)
    cidx = jnp.pad(cidx, ((0, 0), (0, A_pad - A))).reshape(C * A_pad)
    pool_g = _sc_gather(h, cidx)
    return _head(pool_g, C, A, A_pad, W_fc, b_fc, W_out, b_out)


# speedup vs baseline: 2.4245x; 1.1955x over previous
import functools

import jax
import jax.numpy as jnp
from jax import lax
from jax.experimental import pallas as pl
from jax.experimental.pallas import tpu as pltpu
from jax.experimental.pallas import tpu_sc as plsc

EPS = 1e-5


def _softplus(x):
    m = jnp.maximum(x, 0.0)
    return m + jnp.log(jnp.exp(x - m) + jnp.exp(-m))


def _sigmoid(x):
    return 1.0 / (1.0 + jnp.exp(-x))


_CH = 128


@functools.lru_cache(maxsize=None)
def _make_sc_gather(V, D, B):
    info = plsc.get_sparse_core_info()
    NW = info.num_cores * info.num_subcores
    assert B % (8 * NW) == 0 and D % 128 == 0
    b_per_w = B // NW
    ch = min(_CH, b_per_w)
    n_full = b_per_w // ch
    rem = b_per_w - n_full * ch

    mesh = plsc.VectorSubcoreMesh(core_axis_name="c", subcore_axis_name="s")
    scratch = [
        pltpu.VMEM((ch,), jnp.int32),
        pltpu.VMEM((ch, D), jnp.float32),
        pltpu.SemaphoreType.DMA,
    ]
    if rem:
        scratch += [
            pltpu.VMEM((rem,), jnp.int32),
            pltpu.VMEM((rem, D), jnp.float32),
        ]

    @functools.partial(
        pl.kernel, mesh=mesh,
        out_type=jax.ShapeDtypeStruct((B, D), jnp.float32),
        scratch_types=scratch,
    )
    def gather_kernel(table_hbm, idx_hbm, out_hbm, idx_v, rows_v, sem,
                      *rem_scratch):
        wid = lax.axis_index("s") * info.num_cores + lax.axis_index("c")
        base = wid * b_per_w

        def do_chunk(off, idx_ref, rows_ref, size):
            pltpu.sync_copy(idx_hbm.at[pl.ds(off, size)], idx_ref)
            pltpu.async_copy(table_hbm.at[idx_ref], rows_ref, sem).wait()
            pltpu.sync_copy(rows_ref, out_hbm.at[pl.ds(off, size)])

        def body(i, carry):
            do_chunk(base + i * ch, idx_v, rows_v, ch)
            return carry

        lax.fori_loop(0, n_full, body, 0)
        if rem:
            idx_r, rows_r = rem_scratch
            do_chunk(base + n_full * ch, idx_r, rows_r, rem)

    return gather_kernel


def _sc_gather(table, idx):
    V, D = table.shape
    (B,) = idx.shape
    return _make_sc_gather(V, D, B)(table, idx)


def _matmul_body(x_ref, w_ref, b_ref, o_ref):
    o_ref[...] = (
        jnp.dot(x_ref[...], w_ref[...], preferred_element_type=jnp.float32)
        + b_ref[...]
    )


def _matmul(x, W, b, BN=2000):
    N, k = x.shape
    m = W.shape[1]
    return pl.pallas_call(
        _matmul_body,
        grid=(N // BN,),
        in_specs=[
            pl.BlockSpec((BN, k), lambda i: (i, 0)),
            pl.BlockSpec((k, m), lambda i: (0, 0)),
            pl.BlockSpec((1, m), lambda i: (0, 0)),
        ],
        out_specs=pl.BlockSpec((BN, m), lambda i: (i, 0)),
        out_shape=jax.ShapeDtypeStruct((N, m), jnp.float32),
    )(x, W, b.reshape(1, m))


def _pad128_body(x_ref, o_ref):
    x = x_ref[...]
    o_ref[...] = jnp.concatenate(
        [x, jnp.zeros_like(x)], axis=1)


def _pad128(h):
    N, d = h.shape
    BN = 2000
    return pl.pallas_call(
        _pad128_body,
        grid=(N // BN,),
        in_specs=[pl.BlockSpec((BN, d), lambda i: (i, 0))],
        out_specs=pl.BlockSpec((BN, 2 * d), lambda i: (i, 0)),
        out_shape=jax.ShapeDtypeStruct((N, 2 * d), jnp.float32),
    )(h)


def _conv_body(BN, M, h_ref, gg_ref, nf_ref, ws_ref, we_ref, b_ref,
               g2_ref, be2_ref, o_ref):
    h = h_ref[...]
    gg = gg_ref[...]
    nf = nf_ref[...]
    d = h.shape[1]
    dot = functools.partial(jnp.dot, preferred_element_type=jnp.float32)
    s = dot(h, ws_ref[...]) + b_ref[...]
    t = gg + dot(nf, we_ref[...])
    t = t.reshape(BN, M, 2 * d) + s[:, None, :]
    filt = _sigmoid(t[:, :, :d])
    core = _softplus(t[:, :, d:])
    msg = jnp.sum(filt * core, axis=1)
    o_ref[...] = _softplus(h + msg * g2_ref[...] + be2_ref[...])


def _conv_layer(h, gg, nf2, Ws, We, bl, g2l, be2l, BN=400):
    N, d = h.shape
    M = gg.shape[0] // N
    e = nf2.shape[1]
    full = lambda shape: pl.BlockSpec(shape, lambda i: (0, 0))
    return pl.pallas_call(
        functools.partial(_conv_body, BN, M),
        grid=(N // BN,),
        in_specs=[
            pl.BlockSpec((BN, d), lambda i: (i, 0)),
            pl.BlockSpec((BN * M, 2 * d), lambda i: (i, 0)),
            pl.BlockSpec((BN * M, e), lambda i: (i, 0)),
            full((d, 2 * d)), full((e, 2 * d)), full((1, 2 * d)),
            full((1, d)), full((1, d)),
        ],
        out_specs=pl.BlockSpec((BN, d), lambda i: (i, 0)),
        out_shape=jax.ShapeDtypeStruct((N, d), jnp.float32),
    )(h, gg, nf2, Ws, We, bl.reshape(1, 2 * d), g2l.reshape(1, d),
      be2l.reshape(1, d))


def _head_body(A, A_pad, C, pg_ref, wfc_ref, bfc_ref, wout_ref, bout_ref,
               o_ref):
    dp = pg_ref.shape[1]
    pg = pg_ref[...].reshape(C, A_pad, dp)
    valid = lax.broadcasted_iota(jnp.int32, (C, A_pad, dp), 1) < A
    s = jnp.sum(jnp.where(valid, pg, 0.0), axis=1) * (1.0 / A)
    dot = functools.partial(jnp.dot, preferred_element_type=jnp.float32)
    crys = _softplus(dot(s, wfc_ref[...]) + bfc_ref[...])
    o_ref[...] = dot(crys, wout_ref[...]) + bout_ref[...]


def _head(pool_g, C, A, A_pad, W_fc_pad, b_fc, W_out, b_out):
    dp = pool_g.shape[1]
    h_fea = W_fc_pad.shape[1]
    full = lambda shape: pl.BlockSpec(shape, lambda: (0, 0))
    return pl.pallas_call(
        functools.partial(_head_body, A, A_pad, C),
        in_specs=[
            full((C * A_pad, dp)),
            full((dp, h_fea)), full((1, h_fea)),
            full((h_fea, 1)), full((1, 1)),
        ],
        out_specs=full((C, 1)),
        out_shape=jax.ShapeDtypeStruct((C, 1), jnp.float32),
    )(pool_g, W_fc_pad, b_fc.reshape(1, h_fea), W_out, b_out.reshape(1, 1))


def kernel(atom_fea, nbr_fea, nbr_fea_idx, crystal_atom_idx, W_emb, b_emb,
           Wf, bf, g1, be1, g2, be2, W_fc, b_fc, W_out, b_out):
    N, M = nbr_fea_idx.shape
    C, A = crystal_atom_idx.shape
    d = W_emb.shape[1]
    e = nbr_fea.shape[2]
    L = Wf.shape[0]

    inv = 1.0 / jnp.sqrt(1.0 + EPS)
    nf2 = nbr_fea.reshape(N * M, e)
    flat_idx = nbr_fea_idx.reshape(N * M).astype(jnp.int32)

    h = _matmul(atom_fea, W_emb, b_emb)
    zero_d = jnp.zeros((2 * d,), jnp.float32)
    for l in range(L):
        scale = g1[l] * inv
        Wl = Wf[l] * scale[None, :]
        bl = bf[l] * scale + be1[l]
        G = _matmul(h, Wl[d:2 * d], zero_d)
        gg = _sc_gather(G, flat_idx)
        h = _conv_layer(h, gg, nf2, Wl[:d], Wl[2 * d:], bl,
                        g2[l] * inv, be2[l])

    A_pad = 128
    cidx = crystal_atom_idx.astype(jnp.int32)
    cidx = jnp.pad(cidx, ((0, 0), (0, A_pad - A))).reshape(C * A_pad)
    pool_g = _sc_gather(_pad128(h), cidx)
    W_fc_pad = jnp.pad(W_fc, ((0, d), (0, 0)))
    return _head(pool_g, C, A, A_pad, W_fc_pad, b_fc, W_out, b_out)


# speedup vs baseline: 2.6586x; 1.0965x over previous
import functools

import jax
import jax.numpy as jnp
from jax import lax
from jax.experimental import pallas as pl
from jax.experimental.pallas import tpu as pltpu
from jax.experimental.pallas import tpu_sc as plsc

EPS = 1e-5


def _softplus(x):
    m = jnp.maximum(x, 0.0)
    return m + jnp.log(jnp.exp(x - m) + jnp.exp(-m))


def _sigmoid(x):
    return 1.0 / (1.0 + jnp.exp(-x))


_CH = 128


@functools.lru_cache(maxsize=None)
def _make_sc_gather(V, D, B):
    info = plsc.get_sparse_core_info()
    NW = info.num_cores * info.num_subcores
    assert B % (8 * NW) == 0 and D % 128 == 0
    b_per_w = B // NW
    ch = min(_CH, b_per_w)
    n_full = b_per_w // ch
    rem = b_per_w - n_full * ch

    mesh = plsc.VectorSubcoreMesh(core_axis_name="c", subcore_axis_name="s")
    scratch = [
        pltpu.VMEM((ch,), jnp.int32),
        pltpu.VMEM((ch, D), jnp.float32),
        pltpu.SemaphoreType.DMA,
    ]
    if rem:
        scratch += [
            pltpu.VMEM((rem,), jnp.int32),
            pltpu.VMEM((rem, D), jnp.float32),
        ]

    @functools.partial(
        pl.kernel, mesh=mesh,
        out_type=jax.ShapeDtypeStruct((B, D), jnp.float32),
        scratch_types=scratch,
    )
    def gather_kernel(table_hbm, idx_hbm, out_hbm, idx_v, rows_v, sem,
                      *rem_scratch):
        wid = lax.axis_index("s") * info.num_cores + lax.axis_index("c")
        base = wid * b_per_w

        def do_chunk(off, idx_ref, rows_ref, size):
            pltpu.sync_copy(idx_hbm.at[pl.ds(off, size)], idx_ref)
            pltpu.async_copy(table_hbm.at[idx_ref], rows_ref, sem).wait()
            pltpu.sync_copy(rows_ref, out_hbm.at[pl.ds(off, size)])

        def body(i, carry):
            do_chunk(base + i * ch, idx_v, rows_v, ch)
            return carry

        lax.fori_loop(0, n_full, body, 0)
        if rem:
            idx_r, rows_r = rem_scratch
            do_chunk(base + n_full * ch, idx_r, rows_r, rem)

    return gather_kernel


def _sc_gather(table, idx):
    V, D = table.shape
    (B,) = idx.shape
    return _make_sc_gather(V, D, B)(table, idx)


def _matmul_body(x_ref, w_ref, b_ref, o_ref):
    o_ref[...] = (
        jnp.dot(x_ref[...], w_ref[...], preferred_element_type=jnp.float32)
        + b_ref[...]
    )


def _matmul(x, W, b, BN=2000):
    N, k = x.shape
    m = W.shape[1]
    return pl.pallas_call(
        _matmul_body,
        grid=(N // BN,),
        in_specs=[
            pl.BlockSpec((BN, k), lambda i: (i, 0)),
            pl.BlockSpec((k, m), lambda i: (0, 0)),
            pl.BlockSpec((1, m), lambda i: (0, 0)),
        ],
        out_specs=pl.BlockSpec((BN, m), lambda i: (i, 0)),
        out_shape=jax.ShapeDtypeStruct((N, m), jnp.float32),
    )(x, W, b.reshape(1, m))


def _pad128_body(x_ref, o_ref):
    x = x_ref[...]
    o_ref[...] = jnp.concatenate(
        [x, jnp.zeros_like(x)], axis=1)


def _pad128(h):
    N, d = h.shape
    BN = 2000
    return pl.pallas_call(
        _pad128_body,
        grid=(N // BN,),
        in_specs=[pl.BlockSpec((BN, d), lambda i: (i, 0))],
        out_specs=pl.BlockSpec((BN, 2 * d), lambda i: (i, 0)),
        out_shape=jax.ShapeDtypeStruct((N, 2 * d), jnp.float32),
    )(h)


def _conv_body(BN, M, h_ref, gg_ref, nft_ref, ws_ref, we_ref, b_ref,
               g2_ref, be2_ref, o_ref):
    h = h_ref[...]
    gg = gg_ref[...]
    nft = nft_ref[...]
    d = h.shape[1]
    dot = functools.partial(jnp.dot, preferred_element_type=jnp.float32)
    s = dot(h, ws_ref[...]) + b_ref[...]
    t = gg + lax.dot_general(
        nft, we_ref[...], (((0,), (0,)), ((), ())),
        preferred_element_type=jnp.float32)
    t = t.reshape(BN, M, 2 * d) + s[:, None, :]
    filt = _sigmoid(t[:, :, :d])
    core = _softplus(t[:, :, d:])
    msg = jnp.sum(filt * core, axis=1)
    o_ref[...] = _softplus(h + msg * g2_ref[...] + be2_ref[...])


def _conv_layer(h, gg, nft, Ws, We, bl, g2l, be2l, BN=400):
    N, d = h.shape
    M = gg.shape[0] // N
    e = nft.shape[0]
    full = lambda shape: pl.BlockSpec(shape, lambda i: (0, 0))
    return pl.pallas_call(
        functools.partial(_conv_body, BN, M),
        grid=(N // BN,),
        in_specs=[
            pl.BlockSpec((BN, d), lambda i: (i, 0)),
            pl.BlockSpec((BN * M, 2 * d), lambda i: (i, 0)),
            pl.BlockSpec((e, BN * M), lambda i: (0, i)),
            full((d, 2 * d)), full((e, 2 * d)), full((1, 2 * d)),
            full((1, d)), full((1, d)),
        ],
        out_specs=pl.BlockSpec((BN, d), lambda i: (i, 0)),
        out_shape=jax.ShapeDtypeStruct((N, d), jnp.float32),
    )(h, gg, nft, Ws, We, bl.reshape(1, 2 * d), g2l.reshape(1, d),
      be2l.reshape(1, d))


def _head_body(A, A_pad, C, pg_ref, wfc_ref, bfc_ref, wout_ref, bout_ref,
               o_ref):
    dp = pg_ref.shape[1]
    pg = pg_ref[...].reshape(C, A_pad, dp)
    valid = lax.broadcasted_iota(jnp.int32, (C, A_pad, dp), 1) < A
    s = jnp.sum(jnp.where(valid, pg, 0.0), axis=1) * (1.0 / A)
    dot = functools.partial(jnp.dot, preferred_element_type=jnp.float32)
    crys = _softplus(dot(s, wfc_ref[...]) + bfc_ref[...])
    o_ref[...] = dot(crys, wout_ref[...]) + bout_ref[...]


def _head(pool_g, C, A, A_pad, W_fc_pad, b_fc, W_out, b_out):
    dp = pool_g.shape[1]
    h_fea = W_fc_pad.shape[1]
    full = lambda shape: pl.BlockSpec(shape, lambda: (0, 0))
    return pl.pallas_call(
        functools.partial(_head_body, A, A_pad, C),
        in_specs=[
            full((C * A_pad, dp)),
            full((dp, h_fea)), full((1, h_fea)),
            full((h_fea, 1)), full((1, 1)),
        ],
        out_specs=full((C, 1)),
        out_shape=jax.ShapeDtypeStruct((C, 1), jnp.float32),
    )(pool_g, W_fc_pad, b_fc.reshape(1, h_fea), W_out, b_out.reshape(1, 1))


def kernel(atom_fea, nbr_fea, nbr_fea_idx, crystal_atom_idx, W_emb, b_emb,
           Wf, bf, g1, be1, g2, be2, W_fc, b_fc, W_out, b_out):
    N, M = nbr_fea_idx.shape
    C, A = crystal_atom_idx.shape
    d = W_emb.shape[1]
    e = nbr_fea.shape[2]
    L = Wf.shape[0]

    inv = 1.0 / jnp.sqrt(1.0 + EPS)
    nft = nbr_fea.reshape(N * M, e).T
    flat_idx = nbr_fea_idx.reshape(N * M).astype(jnp.int32)

    h = _matmul(atom_fea, W_emb, b_emb)
    zero_d = jnp.zeros((2 * d,), jnp.float32)
    for l in range(L):
        scale = g1[l] * inv
        Wl = Wf[l] * scale[None, :]
        bl = bf[l] * scale + be1[l]
        G = _matmul(h, Wl[d:2 * d], zero_d)
        gg = _sc_gather(G, flat_idx)
        h = _conv_layer(h, gg, nft, Wl[:d], Wl[2 * d:], bl,
                        g2[l] * inv, be2[l])

    A_pad = 128
    cidx = crystal_atom_idx.astype(jnp.int32)
    spread = (jnp.arange(C * (A_pad - A), dtype=jnp.int32) * 97) % N
    cidx = jnp.concatenate(
        [cidx, spread.reshape(C, A_pad - A)], axis=1).reshape(C * A_pad)
    pool_g = _sc_gather(_pad128(h), cidx)
    W_fc_pad = jnp.pad(W_fc, ((0, d), (0, 0)))
    return _head(pool_g, C, A, A_pad, W_fc_pad, b_fc, W_out, b_out)
